# baseline (speedup 1.0000x reference)
import functools

import jax
import jax.numpy as jnp
from jax import lax
from jax.experimental import pallas as pl
from jax.experimental.pallas import tpu as pltpu
from jax.experimental.pallas import tpu_sc as plsc

EOS_COEF = 0.1
_NC, _NS, _L = 2, 16, 16
_LN2 = 0.6931471805599453
_LOG2_COEFS = (0.014598640230194459, -0.07592081220121791,
               0.18865228319232435, -0.3214829482084632,
               0.47172152680207385, -0.7202025944414882,
               1.4426336790038374, 8.121171884499682e-07)


def _dense_body(*refs):
    acc_ref = refs[-1]

    @pl.when(pl.program_id(0) == 0)
    def _init():
        acc_ref[0, 0] = 0.0

    acc = 0.0
    for logits_ref in refs[:-1]:
        c1 = logits_ref.shape[2]
        x = logits_ref[...]
        m = jnp.max(x, axis=-1, keepdims=True)
        se = jnp.sum(jnp.exp(x - m), axis=-1, keepdims=True)
        lse = jnp.log(se) + m
        acc += jnp.sum(lse - x[:, :, c1 - 1:c1])
    acc_ref[0, 0] += acc


def _vi(v):
    return jnp.full((_L,), v, jnp.int32)


def _vf(v):
    return jnp.full((_L,), v, jnp.float32)


def _vlog(s):
    bits = plsc.bitcast(s, jnp.int32)
    e = (bits >> _vi(23)) - _vi(127)
    mant = plsc.bitcast((bits & _vi(0x007FFFFF)) | _vi(0x3F800000),
                        jnp.float32)
    t = mant - _vf(1.0)
    p = _vf(_LOG2_COEFS[0])
    for c in _LOG2_COEFS[1:]:
        p = p * t + _vf(c)
    return (e.astype(jnp.float32) + p) * _vf(_LN2)


def _matched_body(s, num_classes, rpw, logits_hbm, boxes_hbm, tgt_hbm,
                  sizes_hbm, out_hbm, xbuf, pbuf, tbuf, sbuf, obuf):
    c1 = num_classes + 1
    wid = lax.axis_index("s") * _NC + lax.axis_index("c")
    ipw = rpw // s
    b0 = wid * ipw

    for j in range(ipw):
        pltpu.sync_copy(logits_hbm.at[b0 + j], xbuf.at[pl.ds(j * s, s)])
        pltpu.sync_copy(boxes_hbm.at[b0 + j], pbuf.at[pl.ds(j * s, s)])
    pltpu.sync_copy(tgt_hbm.at[pl.ds(wid * rpw, rpw)], tbuf)
    pltpu.sync_copy(sizes_hbm, sbuf)

    lv = lax.iota(jnp.int32, _L)
    corr = jnp.zeros((_L,), jnp.float32)
    wsum = jnp.zeros((_L,), jnp.float32)
    correct = jnp.zeros((_L,), jnp.float32)

    zf = jnp.zeros((_L,), jnp.float32)
    for row0 in range(0, rpw, _L):
        nv = min(_L, rpw - row0)
        ok = lv < _vi(nv)
        rows = jnp.where(ok, _vi(row0) + lv, _vi(0))

        def _p1(c, carry):
            m, am = carry
            vc = jnp.full((_L,), c, jnp.int32)
            v = plsc.load_gather(xbuf, [rows, vc])
            gt = v > m
            return jnp.maximum(m, v), jnp.where(gt, vc, am)

        m, am = lax.fori_loop(0, c1, _p1, (_vf(-jnp.inf), _vi(c1)))

        def _p2(c, se):
            v = plsc.load_gather(xbuf, [rows, jnp.full((_L,), c, jnp.int32)])
            return se + jnp.exp(v - m)

        se = lax.fori_loop(0, c1, _p2, zf)
        lse = _vlog(se) + m

        tci = plsc.load_gather(tbuf, [rows, _vi(4)]).astype(jnp.int32)
        x_tc = plsc.load_gather(xbuf, [rows, tci])
        x_last = plsc.load_gather(xbuf, [rows, _vi(c1 - 1)])
        w = jnp.where(tci == _vi(num_classes), _vf(EOS_COEF), _vf(1.0))
        contrib = w * (lse - x_tc) - _vf(EOS_COEF) * (lse - x_last)
        corr += jnp.where(ok, contrib, zf)
        wsum += jnp.where(ok, w, zf)
        correct += jnp.where(ok & (am == tci), _vf(1.0), zf)

    l1 = jnp.zeros((_L,), jnp.float32)
    for k in range(rpw * 4 // _L):
        e = _vi(k * _L) + lv
        r4 = e >> _vi(2)
        c4 = e & _vi(3)
        pv = plsc.load_gather(pbuf, [r4, c4])
        tv = plsc.load_gather(tbuf, [r4, c4])
        l1 += jnp.abs(pv - tv)

    obuf[0] = corr
    obuf[1] = wsum
    obuf[2] = correct
    obuf[3] = l1
    obuf[4] = jnp.zeros((_L,), jnp.float32)
    obuf[5] = jnp.zeros((_L,), jnp.float32)
    obuf[6] = jnp.zeros((_L,), jnp.float32)
    obuf[7] = jnp.zeros((_L,), jnp.float32)

    @pl.when(wid == 0)
    def _nbox():
        acc = jnp.zeros((_L,), jnp.float32)
        for k in range(sizes_hbm.shape[0] // _L):
            acc += sbuf[pl.ds(k * _L, _L)].astype(jnp.float32)
        obuf[4] = acc

    pltpu.sync_copy(obuf, out_hbm.at[wid])


@jax.jit
def kernel(class_logits, pred_boxes, targets, sizes):
    b, q, c1 = class_logits.shape
    num_classes = c1 - 1
    s = targets.shape[0] // b
    nw = _NC * _NS
    rpw = b * s // nw

    mesh = plsc.VectorSubcoreMesh(core_axis_name="c", subcore_axis_name="s")
    sc = pl.kernel(
        functools.partial(_matched_body, s, num_classes, rpw),
        mesh=mesh,
        out_type=jax.ShapeDtypeStruct((nw, 8, _L), jnp.float32),
        scratch_types=[
            pltpu.VMEM((rpw, c1), jnp.float32),
            pltpu.VMEM((rpw, 4), jnp.float32),
            pltpu.VMEM((rpw, 5), jnp.float32),
            pltpu.VMEM((b,), jnp.int32),
            pltpu.VMEM((8, _L), jnp.float32),
        ],
        compiler_params=pltpu.CompilerParams(use_tc_tiling_on_sc=False,
                                             needs_layout_passes=False),
    )
    parts = sc(class_logits[:, :s], pred_boxes[:, :s], targets, sizes)

    nb, ns = 4, 4
    steps = b // (nb * ns)

    def _mk_map(k):
        return lambda i: (k * steps + i, 0, 0)

    dense = pl.pallas_call(
        _dense_body,
        grid=(steps,),
        in_specs=[pl.BlockSpec((nb, q, c1), _mk_map(k)) for k in range(ns)],
        out_specs=pl.BlockSpec(memory_space=pltpu.SMEM),
        out_shape=jax.ShapeDtypeStruct((1, 1), jnp.float32),
    )(*([class_logits] * ns))[0, 0]

    psum = jnp.sum(parts, axis=(0, 2))
    corr, wsum_m, correct, l1, nbox = (psum[0], psum[1], psum[2], psum[3],
                                       psum[4])
    wnll = EOS_COEF * dense + corr
    sum_w = EOS_COEF * (b * (q - s)) + wsum_m
    loss_ce = wnll / sum_w
    class_error = 100.0 - correct * (100.0 / (b * s))
    loss_bbox = l1 / jnp.maximum(nbox, 1.0)
    return loss_ce, class_error, loss_bbox
